# baseline (speedup 1.0000x reference)
import functools

import jax
import jax.numpy as jnp
from jax import lax
from jax.experimental import pallas as pl
from jax.experimental.pallas import tpu as pltpu
from jax.experimental.pallas import tpu_sc as plsc

N = 10000
D = 128
E = 320000
NC = 2
NS = 16
NW = NC * NS
E_W = 10240
E_PAD = NW * E_W
CHUNK = 64
NBUF = 4
PBUF = 8
TOT_CHUNKS = E_PAD // CHUNK
C0 = 240
C1 = 80
N_PAD = 10112
ROWS_W = N_PAD // NS


def _sc_spmm(h, pk, w4, zeros):
    mesh = plsc.VectorSubcoreMesh(core_axis_name="c", subcore_axis_name="s")

    @functools.partial(
        pl.kernel,
        mesh=mesh,
        out_type=jax.ShapeDtypeStruct((NC, N_PAD, D), jnp.float32),
        scratch_types=[
            pltpu.VMEM((PBUF, 2, CHUNK), jnp.int32),
            pltpu.VMEM((NBUF, CHUNK + 8, D), jnp.float32),
            pltpu.VMEM_SHARED((N_PAD, D), jnp.float32),
            pltpu.SemaphoreType.DMA,
            pltpu.SemaphoreType.DMA,
            pltpu.SemaphoreType.DMA,
            pltpu.SemaphoreType.DMA,
            pltpu.SemaphoreType.DMA,
            pltpu.SemaphoreType.DMA,
            pltpu.SemaphoreType.DMA,
            pltpu.SemaphoreType.DMA,
            pltpu.SemaphoreType.DMA,
            pltpu.SemaphoreType.DMA,
            pltpu.SemaphoreType.DMA,
            pltpu.SemaphoreType.DMA,
            pltpu.SemaphoreType.DMA,
            pltpu.SemaphoreType.DMA,
            pltpu.SemaphoreType.DMA,
            pltpu.SemaphoreType.DMA,
        ],
    )
    def k(h_hbm, pk_hbm, w_hbm, z_hbm, out_hbm,
          pack_v, rows_v, acc,
          c0, c1, c2, c3, c4, c5, c6, c7,
          g0, g1, g2, g3, s0, s1, s2, s3):
        csem = [c0, c1, c2, c3, c4, c5, c6, c7]
        gsem = [g0, g1, g2, g3]
        ssem = [s0, s1, s2, s3]
        cid = lax.axis_index("c")
        sid = lax.axis_index("s")
        rbase = sid * ROWS_W
        nch = jnp.where(cid == 0, C0, C1)
        ngrp = jnp.where(cid == 0, C0 // PBUF, C1 // PBUF)
        base = jnp.where(cid == 0, sid * C0, NS * C0 + sid * C1)

        with jax.named_scope("zero_acc"):
            pltpu.sync_copy(z_hbm.at[pl.ds(rbase, ROWS_W)],
                            acc.at[pl.ds(rbase, ROWS_W)])
            plsc.subcore_barrier()

        def pack_start(c, pb):
            pltpu.async_copy(pk_hbm.at[base + c], pack_v.at[pb], csem[pb])

        def pack_wait(c, pb):
            pltpu.make_async_copy(pk_hbm.at[base + c], pack_v.at[pb],
                                  csem[pb]).wait()

        def gather_start(c, b, pb):
            pltpu.async_copy(h_hbm.at[pack_v.at[pb, 1]],
                             rows_v.at[b, pl.ds(0, CHUNK)], gsem[b])
            pltpu.async_copy(w_hbm.at[base + c],
                             rows_v.at[b, pl.ds(CHUNK, 1)], gsem[b])

        def gather_wait(c, b, pb):
            pltpu.make_async_copy(h_hbm.at[pack_v.at[pb, 1]],
                                  rows_v.at[b, pl.ds(0, CHUNK)],
                                  gsem[b]).wait()
            pltpu.make_async_copy(w_hbm.at[base + c],
                                  rows_v.at[b, pl.ds(CHUNK, 1)],
                                  gsem[b]).wait()

        def scatter_start(c, b, pb):
            pltpu.async_copy(rows_v.at[b, pl.ds(0, CHUNK)],
                             acc.at[pack_v.at[pb, 0]], ssem[b], add=True)

        def scatter_wait(c, b, pb):
            pltpu.make_async_copy(rows_v.at[b, pl.ds(0, CHUNK)],
                                  acc.at[pack_v.at[pb, 0]], ssem[b]).wait()

        def scale(b, pb):
            dnums = lax.GatherDimensionNumbers(
                offset_dims=(), collapsed_slice_dims=(0,),
                start_index_map=(0,))

            def scale_body(g, carry2):
                w16 = rows_v[b, CHUNK, pl.ds(g * 16, 16)]
                for u in range(16):
                    wv = lax.gather(
                        w16, jnp.full((16, 1), u, jnp.int32), dnums, (1,),
                        mode=lax.GatherScatterMode.PROMISE_IN_BOUNDS)
                    e = g * 16 + u
                    for j in range(D // 16):
                        rows_v[b, e, pl.ds(16 * j, 16)] = (
                            rows_v[b, e, pl.ds(16 * j, 16)] * wv)
                return carry2
            lax.fori_loop(0, CHUNK // 16, scale_body, 0)

        for c in range(4):
            pack_start(c, c)
        pack_wait(0, 0)
        pack_wait(1, 1)
        gather_start(0, 0, 0)
        gather_start(1, 1, 1)

        def group_body(grp, carry):
            for k in range(PBUF):
                c = grp * PBUF + k
                b = k % NBUF
                gather_wait(c, b, k)
                scale(b, k)
                scatter_start(c, b, k)

                @pl.when(c >= 2)
                def _():
                    scatter_wait(c - 2, (b + 2) % NBUF, (k + 6) % PBUF)

                @pl.when(c + 4 < nch)
                def _():
                    pack_start(c + 4, (k + 4) % PBUF)

                @pl.when(c + 2 < nch)
                def _():
                    pack_wait(c + 2, (k + 2) % PBUF)
                    gather_start(c + 2, (b + 2) % NBUF, (k + 2) % PBUF)
            return carry
        with jax.named_scope("edge_loop"):
            lax.fori_loop(0, ngrp, group_body, 0)

        with jax.named_scope("drain"):
            scatter_wait(nch - 2, (PBUF - 2) % NBUF, PBUF - 2)
            scatter_wait(nch - 1, (PBUF - 1) % NBUF, PBUF - 1)
            plsc.subcore_barrier()

        with jax.named_scope("writeback"):
            pltpu.sync_copy(acc.at[pl.ds(rbase, ROWS_W)],
                            out_hbm.at[cid, pl.ds(rbase, ROWS_W)])

    return k(h, pk, w4, zeros)


def _tc_linear(x, W, b):
    BLK = 1000

    def body(x_ref, w_ref, b_ref, o_ref):
        o_ref[...] = jnp.dot(x_ref[...], w_ref[...],
                             preferred_element_type=jnp.float32) + b_ref[...]

    return pl.pallas_call(
        body,
        grid=(N // BLK,),
        in_specs=[pl.BlockSpec((BLK, D), lambda i: (i, 0)),
                  pl.BlockSpec((D, D), lambda i: (0, 0)),
                  pl.BlockSpec((1, D), lambda i: (0, 0))],
        out_specs=pl.BlockSpec((BLK, D), lambda i: (i, 0)),
        out_shape=jax.ShapeDtypeStruct((N, D), jnp.float32),
    )(x, W, b.reshape(1, D))


def _tc_combine_linear(p, W, b):
    BLK = 1000

    def body(p_ref, w_ref, b_ref, o_ref):
        hb = jnp.maximum(p_ref[0] + p_ref[1], 0.0)
        o_ref[...] = jnp.dot(hb, w_ref[...],
                             preferred_element_type=jnp.float32) + b_ref[...]

    return pl.pallas_call(
        body,
        grid=(N // BLK,),
        in_specs=[pl.BlockSpec((NC, BLK, D), lambda i: (0, i, 0)),
                  pl.BlockSpec((D, D), lambda i: (0, 0)),
                  pl.BlockSpec((1, D), lambda i: (0, 0))],
        out_specs=pl.BlockSpec((BLK, D), lambda i: (i, 0)),
        out_shape=jax.ShapeDtypeStruct((N, D), jnp.float32),
    )(p, W, b.reshape(1, D))


def _tc_combine(p):
    BLK = 1000

    def body(p_ref, o_ref):
        o_ref[...] = p_ref[0] + p_ref[1]

    return pl.pallas_call(
        body,
        grid=(N // BLK,),
        in_specs=[pl.BlockSpec((NC, BLK, D), lambda i: (0, i, 0))],
        out_specs=pl.BlockSpec((BLK, D), lambda i: (i, 0)),
        out_shape=jax.ShapeDtypeStruct((N, D), jnp.float32),
    )(p)


def kernel(x, edge_index, edge_weight, W1, b1, W2, b2):
    row = edge_index[0].astype(jnp.int32)
    col = edge_index[1].astype(jnp.int32)
    pad = E_PAD - E
    row_p = jnp.concatenate([row, jnp.zeros((pad,), jnp.int32)])
    col_p = jnp.concatenate([col, jnp.zeros((pad,), jnp.int32)])
    w_p = jnp.concatenate([edge_weight.astype(jnp.float32),
                           jnp.zeros((pad,), jnp.float32)])
    pk = jnp.stack([row_p.reshape(TOT_CHUNKS, CHUNK),
                    col_p.reshape(TOT_CHUNKS, CHUNK)], axis=1)
    w4 = jnp.pad(w_p.reshape(TOT_CHUNKS, CHUNK), ((0, 0), (0, D - CHUNK)))
    w4 = w4.reshape(TOT_CHUNKS, 1, D)
    zeros = jnp.zeros((N_PAD, D), jnp.float32)

    h = _tc_linear(x, W1, b1)
    p1 = _sc_spmm(h, pk, w4, zeros)
    h2 = _tc_combine_linear(p1[:, :N], W2, b2)
    p2 = _sc_spmm(h2, pk, w4, zeros)
    return _tc_combine(p2[:, :N])


# speedup vs baseline: 2.7313x; 2.7313x over previous
import functools

import jax
import jax.numpy as jnp
from jax import lax
from jax.experimental import pallas as pl
from jax.experimental.pallas import tpu as pltpu
from jax.experimental.pallas import tpu_sc as plsc

N = 10000
D = 128
E = 320000
NC = 2
NS = 16
NW = NC * NS
E_W = 10240
E_PAD = NW * E_W
CHUNK = 64
NBUF = 4
PBUF = 8
TOT_CHUNKS = E_PAD // CHUNK
C0 = 160
C1 = 160
N_PAD = 10112
ROWS_W = N_PAD // NS


def _sc_spmm(h, pk, w4, zeros):
    mesh = plsc.VectorSubcoreMesh(core_axis_name="c", subcore_axis_name="s")

    @functools.partial(
        pl.kernel,
        mesh=mesh,
        out_type=jax.ShapeDtypeStruct((NC, N_PAD, D), jnp.float32),
        scratch_types=[
            pltpu.VMEM((PBUF, 2, CHUNK), jnp.int32),
            pltpu.VMEM((NBUF, CHUNK + 8, D), jnp.float32),
            pltpu.VMEM_SHARED((N_PAD, D), jnp.float32),
            pltpu.SemaphoreType.DMA,
            pltpu.SemaphoreType.DMA,
            pltpu.SemaphoreType.DMA,
            pltpu.SemaphoreType.DMA,
            pltpu.SemaphoreType.DMA,
            pltpu.SemaphoreType.DMA,
            pltpu.SemaphoreType.DMA,
            pltpu.SemaphoreType.DMA,
            pltpu.SemaphoreType.DMA,
            pltpu.SemaphoreType.DMA,
            pltpu.SemaphoreType.DMA,
            pltpu.SemaphoreType.DMA,
            pltpu.SemaphoreType.DMA,
            pltpu.SemaphoreType.DMA,
            pltpu.SemaphoreType.DMA,
            pltpu.SemaphoreType.DMA,
        ],
    )
    def k(h_hbm, pk_hbm, w_hbm, z_hbm, out_hbm,
          pack_v, rows_v, acc,
          c0, c1, c2, c3, c4, c5, c6, c7,
          g0, g1, g2, g3, s0, s1, s2, s3):
        csem = [c0, c1, c2, c3, c4, c5, c6, c7]
        gsem = [g0, g1, g2, g3]
        ssem = [s0, s1, s2, s3]
        cid = lax.axis_index("c")
        sid = lax.axis_index("s")
        rbase = sid * ROWS_W
        nch = jnp.where(cid == 0, C0, C1)
        ngrp = jnp.where(cid == 0, C0 // PBUF, C1 // PBUF)
        base = jnp.where(cid == 0, sid * C0, NS * C0 + sid * C1)

        with jax.named_scope("zero_acc"):
            pltpu.sync_copy(z_hbm.at[pl.ds(rbase, ROWS_W)],
                            acc.at[pl.ds(rbase, ROWS_W)])
            plsc.subcore_barrier()

        def pack_start(c, pb):
            pltpu.async_copy(pk_hbm.at[base + c], pack_v.at[pb], csem[pb])

        def pack_wait(c, pb):
            pltpu.make_async_copy(pk_hbm.at[base + c], pack_v.at[pb],
                                  csem[pb]).wait()

        def gather_start(c, b, pb):
            pltpu.async_copy(h_hbm.at[pack_v.at[pb, 1]],
                             rows_v.at[b, pl.ds(0, CHUNK)], gsem[b])
            pltpu.async_copy(w_hbm.at[base + c],
                             rows_v.at[b, pl.ds(CHUNK, 1)], gsem[b])

        def gather_wait(c, b, pb):
            pltpu.make_async_copy(h_hbm.at[pack_v.at[pb, 1]],
                                  rows_v.at[b, pl.ds(0, CHUNK)],
                                  gsem[b]).wait()
            pltpu.make_async_copy(w_hbm.at[base + c],
                                  rows_v.at[b, pl.ds(CHUNK, 1)],
                                  gsem[b]).wait()

        def scatter_start(c, b, pb):
            pltpu.async_copy(rows_v.at[b, pl.ds(0, CHUNK)],
                             acc.at[pack_v.at[pb, 0]], ssem[b], add=True)

        def scatter_wait(c, b, pb):
            pltpu.make_async_copy(rows_v.at[b, pl.ds(0, CHUNK)],
                                  acc.at[pack_v.at[pb, 0]], ssem[b]).wait()

        def scale(b, pb):
            dnums = lax.GatherDimensionNumbers(
                offset_dims=(), collapsed_slice_dims=(0,),
                start_index_map=(0,))

            def scale_body(g, carry2):
                w16 = rows_v[b, CHUNK, pl.ds(g * 16, 16)]
                for u in range(16):
                    wv = lax.gather(
                        w16, jnp.full((16, 1), u, jnp.int32), dnums, (1,),
                        mode=lax.GatherScatterMode.PROMISE_IN_BOUNDS)
                    e = g * 16 + u
                    for j in range(D // 16):
                        rows_v[b, e, pl.ds(16 * j, 16)] = (
                            rows_v[b, e, pl.ds(16 * j, 16)] * wv)
                return carry2
            lax.fori_loop(0, CHUNK // 16, scale_body, 0)

        for c in range(4):
            pack_start(c, c)
        pack_wait(0, 0)
        pack_wait(1, 1)
        gather_start(0, 0, 0)
        gather_start(1, 1, 1)

        def group_body(grp, carry):
            for k in range(PBUF):
                c = grp * PBUF + k
                b = k % NBUF
                gather_wait(c, b, k)
                scale(b, k)
                scatter_start(c, b, k)

                @pl.when(c >= 2)
                def _():
                    scatter_wait(c - 2, (b + 2) % NBUF, (k + 6) % PBUF)

                @pl.when(c + 4 < nch)
                def _():
                    pack_start(c + 4, (k + 4) % PBUF)

                @pl.when(c + 2 < nch)
                def _():
                    pack_wait(c + 2, (k + 2) % PBUF)
                    gather_start(c + 2, (b + 2) % NBUF, (k + 2) % PBUF)
            return carry
        with jax.named_scope("edge_loop"):
            lax.fori_loop(0, ngrp, group_body, 0)

        with jax.named_scope("drain"):
            scatter_wait(nch - 2, (PBUF - 2) % NBUF, PBUF - 2)
            scatter_wait(nch - 1, (PBUF - 1) % NBUF, PBUF - 1)
            plsc.subcore_barrier()

        with jax.named_scope("writeback"):
            pltpu.sync_copy(acc.at[pl.ds(rbase, ROWS_W)],
                            out_hbm.at[cid, pl.ds(rbase, ROWS_W)])

    return k(h, pk, w4, zeros)


def _tc_linear(x, W, b):
    BLK = 1000

    def body(x_ref, w_ref, b_ref, o_ref):
        o_ref[...] = jnp.dot(x_ref[...], w_ref[...],
                             preferred_element_type=jnp.float32) + b_ref[...]

    return pl.pallas_call(
        body,
        grid=(N // BLK,),
        in_specs=[pl.BlockSpec((BLK, D), lambda i: (i, 0)),
                  pl.BlockSpec((D, D), lambda i: (0, 0)),
                  pl.BlockSpec((1, D), lambda i: (0, 0))],
        out_specs=pl.BlockSpec((BLK, D), lambda i: (i, 0)),
        out_shape=jax.ShapeDtypeStruct((N, D), jnp.float32),
    )(x, W, b.reshape(1, D))


def _tc_combine_linear(p, W, b):
    BLK = 1000

    def body(p_ref, w_ref, b_ref, o_ref):
        hb = jnp.maximum(p_ref[0] + p_ref[1], 0.0)
        o_ref[...] = jnp.dot(hb, w_ref[...],
                             preferred_element_type=jnp.float32) + b_ref[...]

    return pl.pallas_call(
        body,
        grid=(N // BLK,),
        in_specs=[pl.BlockSpec((NC, BLK, D), lambda i: (0, i, 0)),
                  pl.BlockSpec((D, D), lambda i: (0, 0)),
                  pl.BlockSpec((1, D), lambda i: (0, 0))],
        out_specs=pl.BlockSpec((BLK, D), lambda i: (i, 0)),
        out_shape=jax.ShapeDtypeStruct((N, D), jnp.float32),
    )(p, W, b.reshape(1, D))


def _tc_combine(p):
    BLK = 1000

    def body(p_ref, o_ref):
        o_ref[...] = p_ref[0] + p_ref[1]

    return pl.pallas_call(
        body,
        grid=(N // BLK,),
        in_specs=[pl.BlockSpec((NC, BLK, D), lambda i: (0, i, 0))],
        out_specs=pl.BlockSpec((BLK, D), lambda i: (i, 0)),
        out_shape=jax.ShapeDtypeStruct((N, D), jnp.float32),
    )(p)


def kernel(x, edge_index, edge_weight, W1, b1, W2, b2):
    row = edge_index[0].astype(jnp.int32)
    col = edge_index[1].astype(jnp.int32)
    pad = E_PAD - E
    spread = (jnp.arange(pad, dtype=jnp.int32) * 13) % N
    row_p = jnp.concatenate([row, spread])
    col_p = jnp.concatenate([col, spread])
    w_p = jnp.concatenate([edge_weight.astype(jnp.float32),
                           jnp.zeros((pad,), jnp.float32)])
    pk = jnp.stack([row_p.reshape(TOT_CHUNKS, CHUNK),
                    col_p.reshape(TOT_CHUNKS, CHUNK)], axis=1)
    w4 = jnp.pad(w_p.reshape(TOT_CHUNKS, CHUNK), ((0, 0), (0, D - CHUNK)))
    w4 = w4.reshape(TOT_CHUNKS, 1, D)
    zeros = jnp.zeros((N_PAD, D), jnp.float32)

    h = _tc_linear(x, W1, b1)
    p1 = _sc_spmm(h, pk, w4, zeros)
    h2 = _tc_combine_linear(p1[:, :N], W2, b2)
    p2 = _sc_spmm(h2, pk, w4, zeros)
    return _tc_combine(p2[:, :N])


# speedup vs baseline: 2.8714x; 1.0513x over previous
import functools

import jax
import jax.numpy as jnp
from jax import lax
from jax.experimental import pallas as pl
from jax.experimental.pallas import tpu as pltpu
from jax.experimental.pallas import tpu_sc as plsc

N = 10000
D = 128
E = 320000
NC = 2
NS = 16
NW = NC * NS
CHUNK = 64
NBUF = 4
PBUF = 8
TOT_CHUNKS = E // CHUNK
CW = 160
CLAST = TOT_CHUNKS - 31 * CW
N_PAD = 10112
ROWS_W = N_PAD // NS


def _sc_spmm(h, pk, w4, zeros):
    mesh = plsc.VectorSubcoreMesh(core_axis_name="c", subcore_axis_name="s")

    @functools.partial(
        pl.kernel,
        mesh=mesh,
        out_type=jax.ShapeDtypeStruct((NC, N_PAD, D), jnp.float32),
        scratch_types=[
            pltpu.VMEM((PBUF, 2, CHUNK), jnp.int32),
            pltpu.VMEM((NBUF, CHUNK + 8, D), jnp.float32),
            pltpu.VMEM_SHARED((N_PAD, D), jnp.float32),
            pltpu.SemaphoreType.DMA,
            pltpu.SemaphoreType.DMA,
            pltpu.SemaphoreType.DMA,
            pltpu.SemaphoreType.DMA,
            pltpu.SemaphoreType.DMA,
            pltpu.SemaphoreType.DMA,
            pltpu.SemaphoreType.DMA,
            pltpu.SemaphoreType.DMA,
            pltpu.SemaphoreType.DMA,
            pltpu.SemaphoreType.DMA,
            pltpu.SemaphoreType.DMA,
            pltpu.SemaphoreType.DMA,
            pltpu.SemaphoreType.DMA,
            pltpu.SemaphoreType.DMA,
            pltpu.SemaphoreType.DMA,
            pltpu.SemaphoreType.DMA,
        ],
    )
    def k(h_hbm, pk_hbm, w_hbm, z_hbm, out_hbm,
          pack_v, rows_v, acc,
          c0, c1, c2, c3, c4, c5, c6, c7,
          g0, g1, g2, g3, s0, s1, s2, s3):
        csem = [c0, c1, c2, c3, c4, c5, c6, c7]
        gsem = [g0, g1, g2, g3]
        ssem = [s0, s1, s2, s3]
        cid = lax.axis_index("c")
        sid = lax.axis_index("s")
        rbase = sid * ROWS_W
        wrk = cid * NS + sid
        base = wrk * CW
        nch = jnp.where(wrk == NW - 1, CLAST, CW)
        ngrp = nch // PBUF

        with jax.named_scope("zero_acc"):
            pltpu.sync_copy(z_hbm.at[pl.ds(rbase, ROWS_W)],
                            acc.at[pl.ds(rbase, ROWS_W)])
            plsc.subcore_barrier()

        def pack_start(c, pb):
            pltpu.async_copy(pk_hbm.at[base + c], pack_v.at[pb], csem[pb])

        def pack_wait(c, pb):
            pltpu.make_async_copy(pk_hbm.at[base + c], pack_v.at[pb],
                                  csem[pb]).wait()

        def gather_start(c, b, pb):
            pltpu.async_copy(h_hbm.at[pack_v.at[pb, 1]],
                             rows_v.at[b, pl.ds(0, CHUNK)], gsem[b])
            pltpu.async_copy(w_hbm.at[base + c, 0],
                             rows_v.at[b, CHUNK, pl.ds(0, CHUNK)], gsem[b])

        def gather_wait(c, b, pb):
            pltpu.make_async_copy(h_hbm.at[pack_v.at[pb, 1]],
                                  rows_v.at[b, pl.ds(0, CHUNK)],
                                  gsem[b]).wait()
            pltpu.make_async_copy(w_hbm.at[base + c, 0],
                                  rows_v.at[b, CHUNK, pl.ds(0, CHUNK)],
                                  gsem[b]).wait()

        def scatter_start(c, b, pb):
            pltpu.async_copy(rows_v.at[b, pl.ds(0, CHUNK)],
                             acc.at[pack_v.at[pb, 0]], ssem[b], add=True)

        def scatter_wait(c, b, pb):
            pltpu.make_async_copy(rows_v.at[b, pl.ds(0, CHUNK)],
                                  acc.at[pack_v.at[pb, 0]], ssem[b]).wait()

        def scale(b, pb):
            dnums = lax.GatherDimensionNumbers(
                offset_dims=(), collapsed_slice_dims=(0,),
                start_index_map=(0,))

            def scale_body(g, carry2):
                w16 = rows_v[b, CHUNK, pl.ds(g * 16, 16)]
                for u in range(16):
                    wv = lax.gather(
                        w16, jnp.full((16, 1), u, jnp.int32), dnums, (1,),
                        mode=lax.GatherScatterMode.PROMISE_IN_BOUNDS)
                    e = g * 16 + u
                    for j in range(D // 16):
                        rows_v[b, e, pl.ds(16 * j, 16)] = (
                            rows_v[b, e, pl.ds(16 * j, 16)] * wv)
                return carry2
            lax.fori_loop(0, CHUNK // 16, scale_body, 0)

        for c in range(4):
            pack_start(c, c)
        pack_wait(0, 0)
        pack_wait(1, 1)
        gather_start(0, 0, 0)
        gather_start(1, 1, 1)

        def group_body(grp, carry):
            for k in range(PBUF):
                c = grp * PBUF + k
                b = k % NBUF
                gather_wait(c, b, k)
                scale(b, k)
                scatter_start(c, b, k)

                @pl.when(c >= 2)
                def _():
                    scatter_wait(c - 2, (b + 2) % NBUF, (k + 6) % PBUF)

                @pl.when(c + 4 < nch)
                def _():
                    pack_start(c + 4, (k + 4) % PBUF)

                @pl.when(c + 2 < nch)
                def _():
                    pack_wait(c + 2, (k + 2) % PBUF)
                    gather_start(c + 2, (b + 2) % NBUF, (k + 2) % PBUF)
            return carry
        with jax.named_scope("edge_loop"):
            lax.fori_loop(0, ngrp, group_body, 0)

        with jax.named_scope("drain"):
            scatter_wait(nch - 2, (PBUF - 2) % NBUF, PBUF - 2)
            scatter_wait(nch - 1, (PBUF - 1) % NBUF, PBUF - 1)
            plsc.subcore_barrier()

        with jax.named_scope("writeback"):
            pltpu.sync_copy(acc.at[pl.ds(rbase, ROWS_W)],
                            out_hbm.at[cid, pl.ds(rbase, ROWS_W)])

    return k(h, pk, w4, zeros)


def _tc_linear(x, W, b):
    BLK = 1000

    def body(x_ref, w_ref, b_ref, o_ref):
        o_ref[...] = jnp.dot(x_ref[...], w_ref[...],
                             preferred_element_type=jnp.float32) + b_ref[...]

    return pl.pallas_call(
        body,
        grid=(N // BLK,),
        in_specs=[pl.BlockSpec((BLK, D), lambda i: (i, 0)),
                  pl.BlockSpec((D, D), lambda i: (0, 0)),
                  pl.BlockSpec((1, D), lambda i: (0, 0))],
        out_specs=pl.BlockSpec((BLK, D), lambda i: (i, 0)),
        out_shape=jax.ShapeDtypeStruct((N, D), jnp.float32),
    )(x, W, b.reshape(1, D))


def _tc_combine_linear(p, W, b):
    BLK = 1000

    def body(p_ref, w_ref, b_ref, o_ref):
        hb = jnp.maximum(p_ref[0] + p_ref[1], 0.0)
        o_ref[...] = jnp.dot(hb, w_ref[...],
                             preferred_element_type=jnp.float32) + b_ref[...]

    return pl.pallas_call(
        body,
        grid=(N // BLK,),
        in_specs=[pl.BlockSpec((NC, BLK, D), lambda i: (0, i, 0)),
                  pl.BlockSpec((D, D), lambda i: (0, 0)),
                  pl.BlockSpec((1, D), lambda i: (0, 0))],
        out_specs=pl.BlockSpec((BLK, D), lambda i: (i, 0)),
        out_shape=jax.ShapeDtypeStruct((N, D), jnp.float32),
    )(p, W, b.reshape(1, D))


def _tc_combine(p):
    BLK = 1000

    def body(p_ref, o_ref):
        o_ref[...] = p_ref[0] + p_ref[1]

    return pl.pallas_call(
        body,
        grid=(N // BLK,),
        in_specs=[pl.BlockSpec((NC, BLK, D), lambda i: (0, i, 0))],
        out_specs=pl.BlockSpec((BLK, D), lambda i: (i, 0)),
        out_shape=jax.ShapeDtypeStruct((N, D), jnp.float32),
    )(p)


def kernel(x, edge_index, edge_weight, W1, b1, W2, b2):
    pk = jnp.swapaxes(edge_index.astype(jnp.int32).reshape(2, TOT_CHUNKS, CHUNK),
                      0, 1)
    w4 = edge_weight.astype(jnp.float32).reshape(TOT_CHUNKS, 1, CHUNK)
    zeros = jnp.zeros((N_PAD, D), jnp.float32)

    h = _tc_linear(x, W1, b1)
    p1 = _sc_spmm(h, pk, w4, zeros)
    h2 = _tc_combine_linear(p1, W2, b2)
    p2 = _sc_spmm(h2, pk, w4, zeros)
    return _tc_combine(p2)


# speedup vs baseline: 2.9046x; 1.0115x over previous
import functools

import jax
import jax.numpy as jnp
from jax import lax
from jax.experimental import pallas as pl
from jax.experimental.pallas import tpu as pltpu
from jax.experimental.pallas import tpu_sc as plsc

N = 10000
D = 128
E = 320000
NC = 2
NS = 16
NW = NC * NS
CHUNK = 64
NBUF = 4
PBUF = 8
TOT_CHUNKS = E // CHUNK
CW = 160
CLAST = TOT_CHUNKS - 31 * CW
N_PAD = 10112
ROWS_W = N_PAD // NS


def _sc_spmm(h, rowr, colr, w4, zeros):
    mesh = plsc.VectorSubcoreMesh(core_axis_name="c", subcore_axis_name="s")

    @functools.partial(
        pl.kernel,
        mesh=mesh,
        out_type=jax.ShapeDtypeStruct((NC, N_PAD, D), jnp.float32),
        scratch_types=[
            pltpu.VMEM((PBUF, 2, CHUNK), jnp.int32),
            pltpu.VMEM((NBUF, CHUNK + 8, D), jnp.float32),
            pltpu.VMEM_SHARED((N_PAD, D), jnp.float32),
            pltpu.SemaphoreType.DMA,
            pltpu.SemaphoreType.DMA,
            pltpu.SemaphoreType.DMA,
            pltpu.SemaphoreType.DMA,
            pltpu.SemaphoreType.DMA,
            pltpu.SemaphoreType.DMA,
            pltpu.SemaphoreType.DMA,
            pltpu.SemaphoreType.DMA,
            pltpu.SemaphoreType.DMA,
            pltpu.SemaphoreType.DMA,
            pltpu.SemaphoreType.DMA,
            pltpu.SemaphoreType.DMA,
            pltpu.SemaphoreType.DMA,
            pltpu.SemaphoreType.DMA,
            pltpu.SemaphoreType.DMA,
            pltpu.SemaphoreType.DMA,
        ],
    )
    def k(h_hbm, row_hbm, col_hbm, w_hbm, z_hbm, out_hbm,
          pack_v, rows_v, acc,
          c0, c1, c2, c3, c4, c5, c6, c7,
          g0, g1, g2, g3, s0, s1, s2, s3):
        csem = [c0, c1, c2, c3, c4, c5, c6, c7]
        gsem = [g0, g1, g2, g3]
        ssem = [s0, s1, s2, s3]
        cid = lax.axis_index("c")
        sid = lax.axis_index("s")
        rbase = sid * ROWS_W
        wrk = cid * NS + sid
        base = wrk * CW
        nch = jnp.where(wrk == NW - 1, CLAST, CW)
        ngrp = nch // PBUF

        with jax.named_scope("zero_acc"):
            pltpu.sync_copy(z_hbm.at[pl.ds(rbase, ROWS_W)],
                            acc.at[pl.ds(rbase, ROWS_W)])
            plsc.subcore_barrier()

        def pack_start(c, pb):
            pltpu.async_copy(row_hbm.at[base + c, 0], pack_v.at[pb, 0],
                             csem[pb])
            pltpu.async_copy(col_hbm.at[base + c, 0], pack_v.at[pb, 1],
                             csem[pb])

        def pack_wait(c, pb):
            pltpu.make_async_copy(row_hbm.at[base + c, 0], pack_v.at[pb, 0],
                                  csem[pb]).wait()
            pltpu.make_async_copy(col_hbm.at[base + c, 0], pack_v.at[pb, 1],
                                  csem[pb]).wait()

        def gather_start(c, b, pb):
            pltpu.async_copy(h_hbm.at[pack_v.at[pb, 1]],
                             rows_v.at[b, pl.ds(0, CHUNK)], gsem[b])
            pltpu.async_copy(w_hbm.at[base + c, 0],
                             rows_v.at[b, CHUNK, pl.ds(0, CHUNK)], gsem[b])

        def gather_wait(c, b, pb):
            pltpu.make_async_copy(h_hbm.at[pack_v.at[pb, 1]],
                                  rows_v.at[b, pl.ds(0, CHUNK)],
                                  gsem[b]).wait()
            pltpu.make_async_copy(w_hbm.at[base + c, 0],
                                  rows_v.at[b, CHUNK, pl.ds(0, CHUNK)],
                                  gsem[b]).wait()

        def scatter_start(c, b, pb):
            pltpu.async_copy(rows_v.at[b, pl.ds(0, CHUNK)],
                             acc.at[pack_v.at[pb, 0]], ssem[b], add=True)

        def scatter_wait(c, b, pb):
            pltpu.make_async_copy(rows_v.at[b, pl.ds(0, CHUNK)],
                                  acc.at[pack_v.at[pb, 0]], ssem[b]).wait()

        def scale(b, pb):
            dnums = lax.GatherDimensionNumbers(
                offset_dims=(), collapsed_slice_dims=(0,),
                start_index_map=(0,))

            def scale_body(g, carry2):
                w16 = rows_v[b, CHUNK, pl.ds(g * 16, 16)]
                for u in range(16):
                    wv = lax.gather(
                        w16, jnp.full((16, 1), u, jnp.int32), dnums, (1,),
                        mode=lax.GatherScatterMode.PROMISE_IN_BOUNDS)
                    e = g * 16 + u
                    for j in range(D // 16):
                        rows_v[b, e, pl.ds(16 * j, 16)] = (
                            rows_v[b, e, pl.ds(16 * j, 16)] * wv)
                return carry2
            lax.fori_loop(0, CHUNK // 16, scale_body, 0)

        for c in range(4):
            pack_start(c, c)
        pack_wait(0, 0)
        pack_wait(1, 1)
        gather_start(0, 0, 0)
        gather_start(1, 1, 1)

        def group_body(grp, carry):
            for k in range(PBUF):
                c = grp * PBUF + k
                b = k % NBUF
                gather_wait(c, b, k)
                scale(b, k)
                scatter_start(c, b, k)

                @pl.when(c >= 2)
                def _():
                    scatter_wait(c - 2, (b + 2) % NBUF, (k + 6) % PBUF)

                @pl.when(c + 4 < nch)
                def _():
                    pack_start(c + 4, (k + 4) % PBUF)

                @pl.when(c + 2 < nch)
                def _():
                    pack_wait(c + 2, (k + 2) % PBUF)
                    gather_start(c + 2, (b + 2) % NBUF, (k + 2) % PBUF)
            return carry
        with jax.named_scope("edge_loop"):
            lax.fori_loop(0, ngrp, group_body, 0)

        with jax.named_scope("drain"):
            scatter_wait(nch - 2, (PBUF - 2) % NBUF, PBUF - 2)
            scatter_wait(nch - 1, (PBUF - 1) % NBUF, PBUF - 1)
            plsc.subcore_barrier()

        with jax.named_scope("writeback"):
            pltpu.sync_copy(acc.at[pl.ds(rbase, ROWS_W)],
                            out_hbm.at[cid, pl.ds(rbase, ROWS_W)])

    return k(h, rowr, colr, w4, zeros)


def _tc_linear(x, W, b):
    BLK = 1000

    def body(x_ref, w_ref, b_ref, o_ref):
        o_ref[...] = jnp.dot(x_ref[...], w_ref[...],
                             preferred_element_type=jnp.float32) + b_ref[...]

    return pl.pallas_call(
        body,
        grid=(N // BLK,),
        in_specs=[pl.BlockSpec((BLK, D), lambda i: (i, 0)),
                  pl.BlockSpec((D, D), lambda i: (0, 0)),
                  pl.BlockSpec((1, D), lambda i: (0, 0))],
        out_specs=pl.BlockSpec((BLK, D), lambda i: (i, 0)),
        out_shape=jax.ShapeDtypeStruct((N, D), jnp.float32),
    )(x, W, b.reshape(1, D))


def _tc_combine_linear(p, W, b):
    BLK = 1000

    def body(p_ref, w_ref, b_ref, o_ref):
        hb = jnp.maximum(p_ref[0] + p_ref[1], 0.0)
        o_ref[...] = jnp.dot(hb, w_ref[...],
                             preferred_element_type=jnp.float32) + b_ref[...]

    return pl.pallas_call(
        body,
        grid=(N // BLK,),
        in_specs=[pl.BlockSpec((NC, BLK, D), lambda i: (0, i, 0)),
                  pl.BlockSpec((D, D), lambda i: (0, 0)),
                  pl.BlockSpec((1, D), lambda i: (0, 0))],
        out_specs=pl.BlockSpec((BLK, D), lambda i: (i, 0)),
        out_shape=jax.ShapeDtypeStruct((N, D), jnp.float32),
    )(p, W, b.reshape(1, D))


def _tc_combine(p):
    BLK = 1000

    def body(p_ref, o_ref):
        o_ref[...] = p_ref[0] + p_ref[1]

    return pl.pallas_call(
        body,
        grid=(N // BLK,),
        in_specs=[pl.BlockSpec((NC, BLK, D), lambda i: (0, i, 0))],
        out_specs=pl.BlockSpec((BLK, D), lambda i: (i, 0)),
        out_shape=jax.ShapeDtypeStruct((N, D), jnp.float32),
    )(p)


def kernel(x, edge_index, edge_weight, W1, b1, W2, b2):
    ei = edge_index.astype(jnp.int32)
    rowr = ei[0].reshape(TOT_CHUNKS, 1, CHUNK)
    colr = ei[1].reshape(TOT_CHUNKS, 1, CHUNK)
    w4 = edge_weight.astype(jnp.float32).reshape(TOT_CHUNKS, 1, CHUNK)
    zeros = jnp.zeros((N_PAD, D), jnp.float32)

    h = _tc_linear(x, W1, b1)
    p1 = _sc_spmm(h, rowr, colr, w4, zeros)
    h2 = _tc_combine_linear(p1, W2, b2)
    p2 = _sc_spmm(h2, rowr, colr, w4, zeros)
    return _tc_combine(p2)


# speedup vs baseline: 3.0266x; 1.0420x over previous
import functools

import jax
import jax.numpy as jnp
from jax import lax
from jax.experimental import pallas as pl
from jax.experimental.pallas import tpu as pltpu
from jax.experimental.pallas import tpu_sc as plsc

N = 10000
D = 128
E = 320000
NC = 2
NS = 16
NW = NC * NS
CHUNK = 64
NBUF = 4
PBUF = 8
TOT_CHUNKS = E // CHUNK
CW = 160
CLAST = TOT_CHUNKS - 31 * CW
N_PAD = 10112
ROWS_W = N_PAD // NS


def _sc_spmm(h, ei4, wflat, zeros):
    mesh = plsc.VectorSubcoreMesh(core_axis_name="c", subcore_axis_name="s")

    @functools.partial(
        pl.kernel,
        mesh=mesh,
        out_type=jax.ShapeDtypeStruct((NC, N_PAD, D), jnp.float32),
        scratch_types=[
            pltpu.VMEM((PBUF, 2, CHUNK), jnp.int32),
            pltpu.VMEM((NBUF, CHUNK + 8, D), jnp.float32),
            pltpu.VMEM_SHARED((N_PAD, D), jnp.float32),
            pltpu.SemaphoreType.DMA,
            pltpu.SemaphoreType.DMA,
            pltpu.SemaphoreType.DMA,
            pltpu.SemaphoreType.DMA,
            pltpu.SemaphoreType.DMA,
            pltpu.SemaphoreType.DMA,
            pltpu.SemaphoreType.DMA,
            pltpu.SemaphoreType.DMA,
            pltpu.SemaphoreType.DMA,
            pltpu.SemaphoreType.DMA,
            pltpu.SemaphoreType.DMA,
            pltpu.SemaphoreType.DMA,
            pltpu.SemaphoreType.DMA,
            pltpu.SemaphoreType.DMA,
            pltpu.SemaphoreType.DMA,
            pltpu.SemaphoreType.DMA,
        ],
    )
    def k(h_hbm, ei_hbm, w_hbm, z_hbm, out_hbm,
          pack_v, rows_v, acc,
          c0, c1, c2, c3, c4, c5, c6, c7,
          g0, g1, g2, g3, s0, s1, s2, s3):
        csem = [c0, c1, c2, c3, c4, c5, c6, c7]
        gsem = [g0, g1, g2, g3]
        ssem = [s0, s1, s2, s3]
        cid = lax.axis_index("c")
        sid = lax.axis_index("s")
        rbase = sid * ROWS_W
        wrk = cid * NS + sid
        base = wrk * CW
        nch = jnp.where(wrk == NW - 1, CLAST, CW)
        ngrp = nch // PBUF

        with jax.named_scope("zero_acc"):
            pltpu.sync_copy(z_hbm.at[pl.ds(rbase, ROWS_W)],
                            acc.at[pl.ds(rbase, ROWS_W)])
            plsc.subcore_barrier()

        def pack_start(c, pb):
            pltpu.async_copy(ei_hbm.at[0, base + c, 0], pack_v.at[pb, 0],
                             csem[pb])
            pltpu.async_copy(ei_hbm.at[1, base + c, 0], pack_v.at[pb, 1],
                             csem[pb])

        def pack_wait(c, pb):
            pltpu.make_async_copy(ei_hbm.at[0, base + c, 0], pack_v.at[pb, 0],
                                  csem[pb]).wait()
            pltpu.make_async_copy(ei_hbm.at[1, base + c, 0], pack_v.at[pb, 1],
                                  csem[pb]).wait()

        def gather_start(c, b, pb):
            pltpu.async_copy(h_hbm.at[pack_v.at[pb, 1]],
                             rows_v.at[b, pl.ds(0, CHUNK)], gsem[b])
            pltpu.async_copy(w_hbm.at[pl.ds((base + c) * CHUNK, CHUNK)],
                             rows_v.at[b, CHUNK, pl.ds(0, CHUNK)], gsem[b])

        def gather_wait(c, b, pb):
            pltpu.make_async_copy(h_hbm.at[pack_v.at[pb, 1]],
                                  rows_v.at[b, pl.ds(0, CHUNK)],
                                  gsem[b]).wait()
            pltpu.make_async_copy(w_hbm.at[pl.ds((base + c) * CHUNK, CHUNK)],
                                  rows_v.at[b, CHUNK, pl.ds(0, CHUNK)],
                                  gsem[b]).wait()

        def scatter_start(c, b, pb):
            pltpu.async_copy(rows_v.at[b, pl.ds(0, CHUNK)],
                             acc.at[pack_v.at[pb, 0]], ssem[b], add=True)

        def scatter_wait(c, b, pb):
            pltpu.make_async_copy(rows_v.at[b, pl.ds(0, CHUNK)],
                                  acc.at[pack_v.at[pb, 0]], ssem[b]).wait()

        def scale(b, pb):
            dnums = lax.GatherDimensionNumbers(
                offset_dims=(), collapsed_slice_dims=(0,),
                start_index_map=(0,))

            def scale_body(g, carry2):
                w16 = rows_v[b, CHUNK, pl.ds(g * 16, 16)]
                for u in range(16):
                    wv = lax.gather(
                        w16, jnp.full((16, 1), u, jnp.int32), dnums, (1,),
                        mode=lax.GatherScatterMode.PROMISE_IN_BOUNDS)
                    e = g * 16 + u
                    for j in range(D // 16):
                        rows_v[b, e, pl.ds(16 * j, 16)] = (
                            rows_v[b, e, pl.ds(16 * j, 16)] * wv)
                return carry2
            lax.fori_loop(0, CHUNK // 16, scale_body, 0)

        for c in range(4):
            pack_start(c, c)
        pack_wait(0, 0)
        pack_wait(1, 1)
        gather_start(0, 0, 0)
        gather_start(1, 1, 1)

        def group_body(grp, carry):
            for k in range(PBUF):
                c = grp * PBUF + k
                b = k % NBUF
                gather_wait(c, b, k)
                scale(b, k)
                scatter_start(c, b, k)

                @pl.when(c >= 2)
                def _():
                    scatter_wait(c - 2, (b + 2) % NBUF, (k + 6) % PBUF)

                @pl.when(c + 4 < nch)
                def _():
                    pack_start(c + 4, (k + 4) % PBUF)

                @pl.when(c + 2 < nch)
                def _():
                    pack_wait(c + 2, (k + 2) % PBUF)
                    gather_start(c + 2, (b + 2) % NBUF, (k + 2) % PBUF)
            return carry
        with jax.named_scope("edge_loop"):
            lax.fori_loop(0, ngrp, group_body, 0)

        with jax.named_scope("drain"):
            scatter_wait(nch - 2, (PBUF - 2) % NBUF, PBUF - 2)
            scatter_wait(nch - 1, (PBUF - 1) % NBUF, PBUF - 1)
            plsc.subcore_barrier()

        with jax.named_scope("writeback"):
            pltpu.sync_copy(acc.at[pl.ds(rbase, ROWS_W)],
                            out_hbm.at[cid, pl.ds(rbase, ROWS_W)])

    return k(h, ei4, wflat, zeros)


def _tc_linear(x, W, b):
    BLK = 1000

    def body(x_ref, w_ref, b_ref, o_ref):
        o_ref[...] = jnp.dot(x_ref[...], w_ref[...],
                             preferred_element_type=jnp.float32) + b_ref[...]

    return pl.pallas_call(
        body,
        grid=(N // BLK,),
        in_specs=[pl.BlockSpec((BLK, D), lambda i: (i, 0)),
                  pl.BlockSpec((D, D), lambda i: (0, 0)),
                  pl.BlockSpec((1, D), lambda i: (0, 0))],
        out_specs=pl.BlockSpec((BLK, D), lambda i: (i, 0)),
        out_shape=jax.ShapeDtypeStruct((N, D), jnp.float32),
    )(x, W, b.reshape(1, D))


def _tc_combine_linear(p, W, b):
    BLK = 1000

    def body(p_ref, w_ref, b_ref, o_ref):
        hb = jnp.maximum(p_ref[0] + p_ref[1], 0.0)
        o_ref[...] = jnp.dot(hb, w_ref[...],
                             preferred_element_type=jnp.float32) + b_ref[...]

    return pl.pallas_call(
        body,
        grid=(N // BLK,),
        in_specs=[pl.BlockSpec((NC, BLK, D), lambda i: (0, i, 0)),
                  pl.BlockSpec((D, D), lambda i: (0, 0)),
                  pl.BlockSpec((1, D), lambda i: (0, 0))],
        out_specs=pl.BlockSpec((BLK, D), lambda i: (i, 0)),
        out_shape=jax.ShapeDtypeStruct((N, D), jnp.float32),
    )(p, W, b.reshape(1, D))


def _tc_combine(p):
    BLK = 1000

    def body(p_ref, o_ref):
        o_ref[...] = p_ref[0] + p_ref[1]

    return pl.pallas_call(
        body,
        grid=(N // BLK,),
        in_specs=[pl.BlockSpec((NC, BLK, D), lambda i: (0, i, 0))],
        out_specs=pl.BlockSpec((BLK, D), lambda i: (i, 0)),
        out_shape=jax.ShapeDtypeStruct((N, D), jnp.float32),
    )(p)


def kernel(x, edge_index, edge_weight, W1, b1, W2, b2):
    ei4 = edge_index.astype(jnp.int32).reshape(2, TOT_CHUNKS, 1, CHUNK)
    wflat = edge_weight.astype(jnp.float32)
    zeros = jnp.zeros((N_PAD, D), jnp.float32)

    h = _tc_linear(x, W1, b1)
    p1 = _sc_spmm(h, ei4, wflat, zeros)
    h2 = _tc_combine_linear(p1, W2, b2)
    p2 = _sc_spmm(h2, ei4, wflat, zeros)
    return _tc_combine(p2)
